# baseline (speedup 1.0000x reference)
import functools

import jax
import jax.numpy as jnp
import numpy as np
from jax import lax
from jax.experimental import pallas as pl
from jax.experimental.pallas import tpu as pltpu
from jax.experimental.pallas import tpu_sc as plsc

N = 10000
D = 128
NPAD = 10240
NC = 2
NS = 16
NW = NC * NS
E = 320000
EPW = 10240
EPAD = NW * EPW
CH = 128
NCHUNK = EPW // CH
BR = 1000
NBUF = 2

HALF = 5120
ACCR = HALF + CH
EPT = EPAD // NS
NCH2 = EPT // CH
KSZ = 162 * CH
ZPT = ACCR // NS
WPT2 = 304

_PAD_PACKED = np.asarray(
    (np.arange(EPAD - E) % 240) | ((N + np.arange(EPAD - E) % 240) << 14),
    dtype=np.int32)

_sc_mesh = plsc.VectorSubcoreMesh(core_axis_name="c", subcore_axis_name="s")


@functools.partial(
    pl.kernel,
    out_type=jax.ShapeDtypeStruct((NC, NPAD), jnp.float32),
    mesh=_sc_mesh,
    scratch_types=[
        pltpu.VMEM((NCHUNK, CH), jnp.int32),
        pltpu.VMEM((CH,), jnp.int32),
        pltpu.VMEM((CH,), jnp.float32),
        pltpu.VMEM((NPAD,), jnp.float32),
        pltpu.VMEM_SHARED((NPAD,), jnp.float32),
    ],
)
def _deg_kernel(packed_hbm, ones_hbm, zeros_hbm, out_hbm, pidx, didx, ones_v,
                bounce_v, hist_s):
    c = lax.axis_index("c")
    s = lax.axis_index("s")
    wid = c * NS + s

    @pl.when(s == 0)
    def _():
        pltpu.sync_copy(zeros_hbm, bounce_v)
        pltpu.sync_copy(bounce_v, hist_s)

    pltpu.sync_copy(packed_hbm.at[wid], pidx)
    pltpu.sync_copy(ones_hbm, ones_v)
    plsc.subcore_barrier()

    def chunk(j, carry):
        for l in range(CH // 16):
            didx[pl.ds(l * 16, 16)] = lax.shift_right_logical(
                pidx[j, pl.ds(l * 16, 16)], 14)
        pltpu.sync_copy(ones_v, hist_s.at[didx], add=True)
        return carry

    lax.fori_loop(0, NCHUNK, chunk, 0)
    plsc.subcore_barrier()

    @pl.when(s == 0)
    def _():
        pltpu.sync_copy(hist_s, bounce_v)
        pltpu.sync_copy(bounce_v, out_hbm.at[c])


@functools.partial(
    pl.kernel,
    out_type=jax.ShapeDtypeStruct((N, D), jnp.float32),
    mesh=_sc_mesh,
    compiler_params=pltpu.CompilerParams(needs_layout_passes=False),
    scratch_types=[
        pltpu.VMEM((NCH2, CH), jnp.int32),
        pltpu.VMEM((KSZ,), jnp.int32),
        pltpu.VMEM((CH,), jnp.int32),
        pltpu.VMEM((CH,), jnp.int32),
        pltpu.VMEM((CH,), jnp.int32),
        pltpu.VMEM((CH,), jnp.int32),
        pltpu.VMEM((CH, D), jnp.float32),
        pltpu.VMEM((CH, D), jnp.float32),
        pltpu.VMEM_SHARED((ACCR, D), jnp.float32),
        pltpu.SemaphoreType.DMA,
        pltpu.SemaphoreType.DMA,
    ],
)
def _agg_kernel(g_hbm, packed_hbm, zk_hbm, zrows_hbm, out_hbm, pidx, kept,
                s0, s1, d0, d1, r0, r1, acc, m0, m1):
    sidx = (s0, s1)
    didx = (d0, d1)
    rows = (r0, r1)
    sems = (m0, m1)
    c = lax.axis_index("c")
    s = lax.axis_index("s")
    lo14 = c * (HALF << 14)
    hi = jnp.where(c == 0, HALF, 1 << 14)

    pltpu.sync_copy(packed_hbm.at[s], pidx)
    pltpu.sync_copy(zk_hbm, kept)
    pltpu.sync_copy(zrows_hbm, rows[0])
    for k in range(3):
        rb = min(k * CH, ZPT - CH)
        pltpu.sync_copy(rows[0], acc.at[pl.ds(s * ZPT + rb, CH)])
    plsc.subcore_barrier()

    lo = c * HALF
    iota = lax.iota(jnp.int32, 16)
    shidx = [jnp.maximum(iota - (1 << t), 0) for t in range(4)]

    def prefix(v):
        for t in range(4):
            g = v.at[shidx[t]].get(mode="promise_in_bounds")
            v = v + jnp.where(iota >= (1 << t), g, 0)
        return v

    def fchunk(j, wptr):
        for l in range(CH // 16):
            v = pidx[j, pl.ds(l * 16, 16)]
            d = lax.shift_right_logical(v, 14)
            m = (d >= lo) & (d < hi)
            mi = m.astype(jnp.int32)
            csum = prefix(mi)
            plsc.addupdate_scatter(kept, [wptr + csum - mi], v - lo14, mask=m)
            wptr = wptr + plsc.all_reduce_population_count(m)
        return wptr

    wsplat = lax.fori_loop(0, NCH2, fchunk, jnp.zeros((16,), jnp.int32))
    for t in range(16):
        tv = (HALF + ((iota + 16 * t) & (CH - 1))) << 14
        plsc.addupdate_scatter(kept, [wsplat + iota + 16 * t], tv)
    nkept = wsplat[0]
    nch = lax.shift_right_logical(nkept + (CH - 1), 7)
    tot = jnp.maximum(nch + (nch & 1), NBUF)

    def unpack(j, k):
        for l in range(CH // 16):
            v = kept[pl.ds(j * CH + l * 16, 16)]
            sidx[k][pl.ds(l * 16, 16)] = v & 0x3FFF
            didx[k][pl.ds(l * 16, 16)] = lax.shift_right_logical(v, 14)

    for k in range(NBUF):
        unpack(k, k)
        pltpu.async_copy(g_hbm.at[sidx[k]], rows[k], sems[k])

    def pair(q, carry):
        j0 = q * NBUF
        for k in range(NBUF):
            j = j0 + k
            pltpu.make_async_copy(g_hbm.at[sidx[k]], rows[k], sems[k]).wait()
            pltpu.sync_copy(rows[k], acc.at[didx[k]], add=True)

            @pl.when(j + NBUF < tot)
            def _():
                unpack(j + NBUF, k)
                pltpu.async_copy(g_hbm.at[sidx[k]], rows[k], sems[k])

        return carry

    lax.fori_loop(0, tot // NBUF, pair, 0)
    plsc.subcore_barrier()

    rows_s = jnp.where(s == NS - 1,
                       jnp.where(c == 0, HALF, N - HALF) - (NS - 1) * WPT2,
                       WPT2)
    for k in range(5):
        rb = pl.multiple_of(s * WPT2 + jnp.minimum(k * CH, rows_s - CH), 8)
        pltpu.sync_copy(acc.at[pl.ds(rb, CH)], rows[k % NBUF])
        pltpu.sync_copy(rows[k % NBUF], out_hbm.at[pl.ds(c * HALF + rb, CH), :])


def _mm_body(x_ref, w_ref, out_ref):
    out_ref[...] = jnp.dot(x_ref[...], w_ref[...],
                           preferred_element_type=jnp.float32)


def _tc_mm(x, W):
    return pl.pallas_call(
        _mm_body,
        grid=(N // BR,),
        in_specs=[
            pl.BlockSpec((BR, D), lambda i: (i, 0)),
            pl.BlockSpec((D, D), lambda i: (0, 0)),
        ],
        out_specs=pl.BlockSpec((BR, D), lambda i: (i, 0)),
        out_shape=jax.ShapeDtypeStruct((N, D), jnp.float32),
    )(x, W)


def _scale_body(cnt_ref, u_ref, g_ref, dis_ref):
    dis = lax.rsqrt(cnt_ref[0] + cnt_ref[1] + 1.0)
    g_ref[...] = u_ref[...] * dis
    dis_ref[...] = dis


def _tc_scale(cnt3, u1):
    return pl.pallas_call(
        _scale_body,
        grid=(N // BR,),
        in_specs=[
            pl.BlockSpec((NC, BR, 1), lambda i: (0, i, 0)),
            pl.BlockSpec((BR, D), lambda i: (i, 0)),
        ],
        out_specs=[
            pl.BlockSpec((BR, D), lambda i: (i, 0)),
            pl.BlockSpec((BR, 1), lambda i: (i, 0)),
        ],
        out_shape=[
            jax.ShapeDtypeStruct((N, D), jnp.float32),
            jax.ShapeDtypeStruct((N, 1), jnp.float32),
        ],
    )(cnt3, u1)


def _tc2_body(a_ref, g_ref, dis_ref, bias_ref, w_ref, out_ref):
    dis = dis_ref[...]
    h = jnp.maximum(dis * (a_ref[...] + g_ref[...]) + bias_ref[...], 0.0)
    out_ref[...] = jnp.dot(h, w_ref[...],
                           preferred_element_type=jnp.float32) * dis


def _tc2(acc, g1, dis_col, b1, W2):
    return pl.pallas_call(
        _tc2_body,
        grid=(N // BR,),
        in_specs=[
            pl.BlockSpec((BR, D), lambda i: (i, 0)),
            pl.BlockSpec((BR, D), lambda i: (i, 0)),
            pl.BlockSpec((BR, 1), lambda i: (i, 0)),
            pl.BlockSpec((1, D), lambda i: (0, 0)),
            pl.BlockSpec((D, D), lambda i: (0, 0)),
        ],
        out_specs=pl.BlockSpec((BR, D), lambda i: (i, 0)),
        out_shape=jax.ShapeDtypeStruct((N, D), jnp.float32),
    )(acc, g1, dis_col, b1, W2)


def _tc3_body(a_ref, g_ref, dis_ref, bias_ref, out_ref):
    out_ref[...] = jnp.maximum(
        dis_ref[...] * (a_ref[...] + g_ref[...]) + bias_ref[...], 0.0)


def _tc3(acc, g2, dis_col, b2):
    return pl.pallas_call(
        _tc3_body,
        grid=(N // BR,),
        in_specs=[
            pl.BlockSpec((BR, D), lambda i: (i, 0)),
            pl.BlockSpec((BR, D), lambda i: (i, 0)),
            pl.BlockSpec((BR, 1), lambda i: (i, 0)),
            pl.BlockSpec((1, D), lambda i: (0, 0)),
        ],
        out_specs=pl.BlockSpec((BR, D), lambda i: (i, 0)),
        out_shape=jax.ShapeDtypeStruct((N, D), jnp.float32),
    )(acc, g2, dis_col, b2)


@jax.jit
def kernel(x, edge_index, W1, b1, W2, b2):
    packed = jnp.concatenate(
        [edge_index[0] | (edge_index[1] << 14), jnp.asarray(_PAD_PACKED)])
    packed_ws = packed.reshape(NW, NCHUNK, CH)
    packed_ts = packed.reshape(NS, NCH2, CH)

    ones_ch = jnp.ones((CH,), jnp.float32)
    zeros_hist = jnp.zeros((NPAD,), jnp.float32)
    zeros_rows = jnp.zeros((CH, D), jnp.float32)
    zeros_kept = jnp.zeros((KSZ,), jnp.int32)

    u1 = _tc_mm(x, W1)
    cnt = _deg_kernel(packed_ws, ones_ch, zeros_hist)
    g1, dis_col = _tc_scale(cnt[:, :N].reshape(NC, N, 1), u1)
    acc1 = _agg_kernel(g1, packed_ts, zeros_kept, zeros_rows)
    g2 = _tc2(acc1, g1, dis_col, b1.reshape(1, D), W2)
    acc2 = _agg_kernel(g2, packed_ts, zeros_kept, zeros_rows)
    return _tc3(acc2, g2, dis_col, b2.reshape(1, D))


# speedup vs baseline: 2.1567x; 2.1567x over previous
import functools

import jax
import jax.numpy as jnp
import numpy as np
from jax import lax
from jax.experimental import pallas as pl
from jax.experimental.pallas import tpu as pltpu
from jax.experimental.pallas import tpu_sc as plsc

N = 10000
D = 128
NPAD = 10240
NC = 2
NS = 16
NW = NC * NS
E = 320000
EPW = 10240
EPAD = NW * EPW
CH = 128
NCHUNK = EPW // CH
ZPT = NPAD // NS
WPT = 632
BR = 2000
NBUF = 2

_PAD_PACKED = np.zeros((EPAD,), dtype=np.int32)
_PAD_PACKED[E:] = (np.arange(EPAD - E) % 240) | (
    (N + np.arange(EPAD - E) % 240) << 14)

_sc_mesh = plsc.VectorSubcoreMesh(core_axis_name="c", subcore_axis_name="s")


@functools.partial(
    pl.kernel,
    out_type=jax.ShapeDtypeStruct((NC, NPAD), jnp.float32),
    mesh=_sc_mesh,
    scratch_types=[
        pltpu.VMEM((NCHUNK, CH), jnp.int32),
        pltpu.VMEM((CH,), jnp.int32),
        pltpu.VMEM((CH,), jnp.float32),
        pltpu.VMEM((NPAD,), jnp.float32),
        pltpu.VMEM_SHARED((NPAD,), jnp.float32),
    ],
)
def _deg_kernel(packed_hbm, ones_hbm, zeros_hbm, out_hbm, pidx, didx, ones_v,
                bounce_v, hist_s):
    c = lax.axis_index("c")
    s = lax.axis_index("s")
    wid = c * NS + s

    @pl.when(s == 0)
    def _():
        pltpu.sync_copy(zeros_hbm, bounce_v)
        pltpu.sync_copy(bounce_v, hist_s)

    pltpu.sync_copy(packed_hbm.at[wid], pidx)
    pltpu.sync_copy(ones_hbm, ones_v)
    plsc.subcore_barrier()

    def chunk(j, carry):
        for l in range(CH // 16):
            didx[pl.ds(l * 16, 16)] = lax.shift_right_logical(
                pidx[j, pl.ds(l * 16, 16)], 14)
        pltpu.sync_copy(ones_v, hist_s.at[didx], add=True)
        return carry

    lax.fori_loop(0, NCHUNK, chunk, 0)
    plsc.subcore_barrier()

    @pl.when(s == 0)
    def _():
        pltpu.sync_copy(hist_s, bounce_v)
        pltpu.sync_copy(bounce_v, out_hbm.at[c])


@functools.partial(
    pl.kernel,
    out_type=jax.ShapeDtypeStruct((NC, N, D), jnp.float32),
    mesh=_sc_mesh,
    scratch_types=[
        pltpu.VMEM((NCHUNK, CH), jnp.int32),
        pltpu.VMEM((CH,), jnp.int32),
        pltpu.VMEM((CH,), jnp.int32),
        pltpu.VMEM((CH,), jnp.int32),
        pltpu.VMEM((CH,), jnp.int32),
        pltpu.VMEM((CH, D), jnp.float32),
        pltpu.VMEM((CH, D), jnp.float32),
        pltpu.VMEM_SHARED((NPAD, D), jnp.float32),
        pltpu.SemaphoreType.DMA,
        pltpu.SemaphoreType.DMA,
    ],
)
def _agg_kernel(g_hbm, packed_hbm, zrows_hbm, out_hbm, pidx,
                s0, s1, d0, d1, r0, r1, acc, m0, m1):
    sidx = (s0, s1)
    didx = (d0, d1)
    rows = (r0, r1)
    sems = (m0, m1)
    c = lax.axis_index("c")
    s = lax.axis_index("s")
    wid = c * NS + s

    def unpack(j, k):
        for l in range(CH // 16):
            v = pidx[j, pl.ds(l * 16, 16)]
            sidx[k][pl.ds(l * 16, 16)] = v & 0x3FFF
            didx[k][pl.ds(l * 16, 16)] = lax.shift_right_logical(v, 14)

    pltpu.sync_copy(packed_hbm.at[wid], pidx)
    pltpu.sync_copy(zrows_hbm, rows[0])
    for k in range(ZPT // CH):
        pltpu.sync_copy(rows[0], acc.at[pl.ds(s * ZPT + k * CH, CH)])
    plsc.subcore_barrier()

    for k in range(NBUF):
        unpack(k, k)
        pltpu.async_copy(g_hbm.at[sidx[k]], rows[k], sems[k])

    def pair(q, carry):
        j0 = q * NBUF
        for k in range(NBUF):
            j = j0 + k
            pltpu.make_async_copy(g_hbm.at[sidx[k]], rows[k], sems[k]).wait()
            pltpu.sync_copy(rows[k], acc.at[didx[k]], add=True)

            @pl.when(j + NBUF < NCHUNK)
            def _():
                unpack(j + NBUF, k)
                pltpu.async_copy(g_hbm.at[sidx[k]], rows[k], sems[k])

        return carry

    lax.fori_loop(0, NCHUNK // NBUF, pair, 0)
    plsc.subcore_barrier()

    rows_s = jnp.where(s == NS - 1, N - (NS - 1) * WPT, WPT)
    base = s * WPT
    for k in range(5):
        rb = pl.multiple_of(base + jnp.minimum(k * CH, rows_s - CH), 8)
        pltpu.sync_copy(acc.at[pl.ds(rb, CH)], rows[k % NBUF])
        pltpu.sync_copy(rows[k % NBUF], out_hbm.at[c, pl.ds(rb, CH), :])


def _mm_body(x_ref, w_ref, out_ref):
    out_ref[...] = jnp.dot(x_ref[...], w_ref[...],
                           preferred_element_type=jnp.float32)


def _tc_mm(x, W):
    return pl.pallas_call(
        _mm_body,
        grid=(N // BR,),
        in_specs=[
            pl.BlockSpec((BR, D), lambda i: (i, 0)),
            pl.BlockSpec((D, D), lambda i: (0, 0)),
        ],
        out_specs=pl.BlockSpec((BR, D), lambda i: (i, 0)),
        out_shape=jax.ShapeDtypeStruct((N, D), jnp.float32),
    )(x, W)


def _scale_body(cnt_ref, u_ref, g_ref, dis_ref):
    dis = lax.rsqrt(cnt_ref[0] + cnt_ref[1] + 1.0)
    g_ref[...] = u_ref[...] * dis
    dis_ref[...] = dis


def _tc_scale(cnt3, u1):
    return pl.pallas_call(
        _scale_body,
        grid=(N // BR,),
        in_specs=[
            pl.BlockSpec((NC, BR, 1), lambda i: (0, i, 0)),
            pl.BlockSpec((BR, D), lambda i: (i, 0)),
        ],
        out_specs=[
            pl.BlockSpec((BR, D), lambda i: (i, 0)),
            pl.BlockSpec((BR, 1), lambda i: (i, 0)),
        ],
        out_shape=[
            jax.ShapeDtypeStruct((N, D), jnp.float32),
            jax.ShapeDtypeStruct((N, 1), jnp.float32),
        ],
    )(cnt3, u1)


def _tc2_body(a_ref, g_ref, dis_ref, bias_ref, w_ref, out_ref):
    dis = dis_ref[...]
    h = jnp.maximum(dis * (a_ref[0] + a_ref[1] + g_ref[...]) + bias_ref[...], 0.0)
    out_ref[...] = jnp.dot(h, w_ref[...],
                           preferred_element_type=jnp.float32) * dis


def _tc2(acc, g1, dis_col, b1, W2):
    return pl.pallas_call(
        _tc2_body,
        grid=(N // BR,),
        in_specs=[
            pl.BlockSpec((NC, BR, D), lambda i: (0, i, 0)),
            pl.BlockSpec((BR, D), lambda i: (i, 0)),
            pl.BlockSpec((BR, 1), lambda i: (i, 0)),
            pl.BlockSpec((1, D), lambda i: (0, 0)),
            pl.BlockSpec((D, D), lambda i: (0, 0)),
        ],
        out_specs=pl.BlockSpec((BR, D), lambda i: (i, 0)),
        out_shape=jax.ShapeDtypeStruct((N, D), jnp.float32),
    )(acc, g1, dis_col, b1, W2)


def _tc3_body(a_ref, g_ref, dis_ref, bias_ref, out_ref):
    out_ref[...] = jnp.maximum(
        dis_ref[...] * (a_ref[0] + a_ref[1] + g_ref[...]) + bias_ref[...], 0.0)


def _tc3(acc, g2, dis_col, b2):
    return pl.pallas_call(
        _tc3_body,
        grid=(N // BR,),
        in_specs=[
            pl.BlockSpec((NC, BR, D), lambda i: (0, i, 0)),
            pl.BlockSpec((BR, D), lambda i: (i, 0)),
            pl.BlockSpec((BR, 1), lambda i: (i, 0)),
            pl.BlockSpec((1, D), lambda i: (0, 0)),
        ],
        out_specs=pl.BlockSpec((BR, D), lambda i: (i, 0)),
        out_shape=jax.ShapeDtypeStruct((N, D), jnp.float32),
    )(acc, g2, dis_col, b2)


@jax.jit
def kernel(x, edge_index, W1, b1, W2, b2):
    packed = jnp.asarray(_PAD_PACKED).at[:E].set(
        edge_index[0] | (edge_index[1] << 14)).reshape(NW, NCHUNK, CH)

    ones_ch = jnp.ones((CH,), jnp.float32)
    zeros_hist = jnp.zeros((NPAD,), jnp.float32)
    zeros_rows = jnp.zeros((CH, D), jnp.float32)

    u1 = _tc_mm(x, W1)
    cnt = _deg_kernel(packed, ones_ch, zeros_hist)
    g1, dis_col = _tc_scale(cnt[:, :N].reshape(NC, N, 1), u1)
    acc1 = _agg_kernel(g1, packed, zeros_rows)
    g2 = _tc2(acc1, g1, dis_col, b1.reshape(1, D), W2)
    acc2 = _agg_kernel(g2, packed, zeros_rows)
    return _tc3(acc2, g2, dis_col, b2.reshape(1, D))


# speedup vs baseline: 2.3062x; 1.0693x over previous
import functools

import jax
import jax.numpy as jnp
import numpy as np
from jax import lax
from jax.experimental import pallas as pl
from jax.experimental.pallas import tpu as pltpu
from jax.experimental.pallas import tpu_sc as plsc

N = 10000
D = 128
NPAD = 10240
NC = 2
NS = 16
NW = NC * NS
E = 320000
CH = 128
NCHT = E // CH
NCHK = 80
LASTC = NCHT - (NW - 1) * NCHK
ZPT = NPAD // NS
WPT = 632
BR = 2000
NBUF = 2

_sc_mesh = plsc.VectorSubcoreMesh(core_axis_name="c", subcore_axis_name="s")


@functools.partial(
    pl.kernel,
    out_type=[
        jax.ShapeDtypeStruct((NC, NPAD), jnp.float32),
        jax.ShapeDtypeStruct((E,), jnp.int32),
    ],
    mesh=_sc_mesh,
    scratch_types=[
        pltpu.VMEM((NCHK * CH,), jnp.int32),
        pltpu.VMEM((NCHK * CH,), jnp.int32),
        pltpu.VMEM((NCHK * CH,), jnp.int32),
        pltpu.VMEM((CH,), jnp.int32),
        pltpu.VMEM((CH,), jnp.float32),
        pltpu.VMEM((NPAD,), jnp.float32),
        pltpu.VMEM_SHARED((NPAD,), jnp.float32),
    ],
)
def _deg_kernel(ei_hbm, ones_hbm, zeros_hbm, cnt_hbm, pk_hbm, sst, dstt, pst,
                didx, ones_v, bounce_v, hist_s):
    c = lax.axis_index("c")
    s = lax.axis_index("s")
    wid = c * NS + s
    base = wid * NCHK * CH

    @pl.when(s == 0)
    def _():
        pltpu.sync_copy(zeros_hbm, bounce_v)
        pltpu.sync_copy(bounce_v, hist_s)

    @pl.when(wid < NW - 1)
    def _():
        pltpu.sync_copy(ei_hbm.at[0, pl.ds(base, NCHK * CH)], sst)
        pltpu.sync_copy(ei_hbm.at[1, pl.ds(base, NCHK * CH)], dstt)

    @pl.when(wid == NW - 1)
    def _():
        pltpu.sync_copy(ei_hbm.at[0, pl.ds(base, LASTC * CH)],
                        sst.at[pl.ds(0, LASTC * CH)])
        pltpu.sync_copy(ei_hbm.at[1, pl.ds(base, LASTC * CH)],
                        dstt.at[pl.ds(0, LASTC * CH)])

    pltpu.sync_copy(ones_hbm, ones_v)
    plsc.subcore_barrier()
    nmine = jnp.where(wid == NW - 1, LASTC, NCHK)

    def chunk(j, carry):
        for l in range(CH // 16):
            dv = dstt[pl.ds(j * CH + l * 16, 16)]
            didx[pl.ds(l * 16, 16)] = dv
            pst[pl.ds(j * CH + l * 16, 16)] = (
                sst[pl.ds(j * CH + l * 16, 16)] | (dv << 14))
        pltpu.sync_copy(ones_v, hist_s.at[didx], add=True)
        return carry

    lax.fori_loop(0, nmine, chunk, 0)

    @pl.when(wid < NW - 1)
    def _():
        pltpu.sync_copy(pst, pk_hbm.at[pl.ds(base, NCHK * CH)])

    @pl.when(wid == NW - 1)
    def _():
        pltpu.sync_copy(pst.at[pl.ds(0, LASTC * CH)],
                        pk_hbm.at[pl.ds(base, LASTC * CH)])

    plsc.subcore_barrier()

    @pl.when(s == 0)
    def _():
        pltpu.sync_copy(hist_s, bounce_v)
        pltpu.sync_copy(bounce_v, cnt_hbm.at[c])


@functools.partial(
    pl.kernel,
    out_type=jax.ShapeDtypeStruct((NC, N, D), jnp.float32),
    mesh=_sc_mesh,
    scratch_types=[
        pltpu.VMEM((NCHK * CH,), jnp.int32),
        pltpu.VMEM((CH,), jnp.int32),
        pltpu.VMEM((CH,), jnp.int32),
        pltpu.VMEM((CH,), jnp.int32),
        pltpu.VMEM((CH,), jnp.int32),
        pltpu.VMEM((CH, D), jnp.float32),
        pltpu.VMEM((CH, D), jnp.float32),
        pltpu.VMEM_SHARED((NPAD, D), jnp.float32),
        pltpu.SemaphoreType.DMA,
        pltpu.SemaphoreType.DMA,
    ],
)
def _agg_kernel(g_hbm, packed_hbm, zrows_hbm, out_hbm, pidx,
                s0, s1, d0, d1, r0, r1, acc, m0, m1):
    sidx = (s0, s1)
    didx = (d0, d1)
    rows = (r0, r1)
    sems = (m0, m1)
    c = lax.axis_index("c")
    s = lax.axis_index("s")
    wid = c * NS + s

    def unpack(j, k):
        for l in range(CH // 16):
            v = pidx[pl.ds(j * CH + l * 16, 16)]
            sidx[k][pl.ds(l * 16, 16)] = v & 0x3FFF
            didx[k][pl.ds(l * 16, 16)] = lax.shift_right_logical(v, 14)

    base = wid * NCHK * CH

    @pl.when(wid < NW - 1)
    def _():
        pltpu.sync_copy(packed_hbm.at[pl.ds(base, NCHK * CH)], pidx)

    @pl.when(wid == NW - 1)
    def _():
        pltpu.sync_copy(packed_hbm.at[pl.ds(base, LASTC * CH)],
                        pidx.at[pl.ds(0, LASTC * CH)])

    pltpu.sync_copy(zrows_hbm, rows[0])
    for k in range(ZPT // CH):
        pltpu.sync_copy(rows[0], acc.at[pl.ds(s * ZPT + k * CH, CH)])
    plsc.subcore_barrier()
    nmine = jnp.where(wid == NW - 1, LASTC, NCHK)

    for k in range(NBUF):
        unpack(k, k)
        pltpu.async_copy(g_hbm.at[sidx[k]], rows[k], sems[k])

    def pair(q, carry):
        j0 = q * NBUF
        for k in range(NBUF):
            j = j0 + k
            pltpu.make_async_copy(g_hbm.at[sidx[k]], rows[k], sems[k]).wait()
            pltpu.sync_copy(rows[k], acc.at[didx[k]], add=True)

            @pl.when(j + NBUF < nmine)
            def _():
                unpack(j + NBUF, k)
                pltpu.async_copy(g_hbm.at[sidx[k]], rows[k], sems[k])

        return carry

    lax.fori_loop(0, nmine // NBUF, pair, 0)
    plsc.subcore_barrier()

    rows_s = jnp.where(s == NS - 1, N - (NS - 1) * WPT, WPT)
    base = s * WPT
    for k in range(5):
        rb = pl.multiple_of(base + jnp.minimum(k * CH, rows_s - CH), 8)
        pltpu.sync_copy(acc.at[pl.ds(rb, CH)], rows[k % NBUF])
        pltpu.sync_copy(rows[k % NBUF], out_hbm.at[c, pl.ds(rb, CH), :])


def _mm_body(x_ref, w_ref, out_ref):
    out_ref[...] = jnp.dot(x_ref[...], w_ref[...],
                           preferred_element_type=jnp.float32)


def _tc_mm(x, W):
    return pl.pallas_call(
        _mm_body,
        grid=(N // BR,),
        in_specs=[
            pl.BlockSpec((BR, D), lambda i: (i, 0)),
            pl.BlockSpec((D, D), lambda i: (0, 0)),
        ],
        out_specs=pl.BlockSpec((BR, D), lambda i: (i, 0)),
        out_shape=jax.ShapeDtypeStruct((N, D), jnp.float32),
    )(x, W)


def _scale_body(cnt_ref, u_ref, g_ref, dis_ref):
    dis = lax.rsqrt(cnt_ref[0] + cnt_ref[1] + 1.0)
    g_ref[...] = u_ref[...] * dis
    dis_ref[...] = dis


def _tc_scale(cnt3, u1):
    return pl.pallas_call(
        _scale_body,
        grid=(N // BR,),
        in_specs=[
            pl.BlockSpec((NC, BR, 1), lambda i: (0, i, 0)),
            pl.BlockSpec((BR, D), lambda i: (i, 0)),
        ],
        out_specs=[
            pl.BlockSpec((BR, D), lambda i: (i, 0)),
            pl.BlockSpec((BR, 1), lambda i: (i, 0)),
        ],
        out_shape=[
            jax.ShapeDtypeStruct((N, D), jnp.float32),
            jax.ShapeDtypeStruct((N, 1), jnp.float32),
        ],
    )(cnt3, u1)


def _tc2_body(a_ref, g_ref, dis_ref, bias_ref, w_ref, out_ref):
    dis = dis_ref[...]
    h = jnp.maximum(dis * (a_ref[0] + a_ref[1] + g_ref[...]) + bias_ref[...], 0.0)
    out_ref[...] = jnp.dot(h, w_ref[...],
                           preferred_element_type=jnp.float32) * dis


def _tc2(acc, g1, dis_col, b1, W2):
    return pl.pallas_call(
        _tc2_body,
        grid=(N // BR,),
        in_specs=[
            pl.BlockSpec((NC, BR, D), lambda i: (0, i, 0)),
            pl.BlockSpec((BR, D), lambda i: (i, 0)),
            pl.BlockSpec((BR, 1), lambda i: (i, 0)),
            pl.BlockSpec((1, D), lambda i: (0, 0)),
            pl.BlockSpec((D, D), lambda i: (0, 0)),
        ],
        out_specs=pl.BlockSpec((BR, D), lambda i: (i, 0)),
        out_shape=jax.ShapeDtypeStruct((N, D), jnp.float32),
    )(acc, g1, dis_col, b1, W2)


def _tc3_body(a_ref, g_ref, dis_ref, bias_ref, out_ref):
    out_ref[...] = jnp.maximum(
        dis_ref[...] * (a_ref[0] + a_ref[1] + g_ref[...]) + bias_ref[...], 0.0)


def _tc3(acc, g2, dis_col, b2):
    return pl.pallas_call(
        _tc3_body,
        grid=(N // BR,),
        in_specs=[
            pl.BlockSpec((NC, BR, D), lambda i: (0, i, 0)),
            pl.BlockSpec((BR, D), lambda i: (i, 0)),
            pl.BlockSpec((BR, 1), lambda i: (i, 0)),
            pl.BlockSpec((1, D), lambda i: (0, 0)),
        ],
        out_specs=pl.BlockSpec((BR, D), lambda i: (i, 0)),
        out_shape=jax.ShapeDtypeStruct((N, D), jnp.float32),
    )(acc, g2, dis_col, b2)


@jax.jit
def kernel(x, edge_index, W1, b1, W2, b2):
    ones_ch = jnp.ones((CH,), jnp.float32)
    zeros_hist = jnp.zeros((NPAD,), jnp.float32)
    zeros_rows = jnp.zeros((CH, D), jnp.float32)

    u1 = _tc_mm(x, W1)
    cnt, packed = _deg_kernel(edge_index, ones_ch, zeros_hist)
    g1, dis_col = _tc_scale(cnt[:, :N].reshape(NC, N, 1), u1)
    acc1 = _agg_kernel(g1, packed, zeros_rows)
    g2 = _tc2(acc1, g1, dis_col, b1.reshape(1, D), W2)
    acc2 = _agg_kernel(g2, packed, zeros_rows)
    return _tc3(acc2, g2, dis_col, b2.reshape(1, D))
